# baseline (speedup 1.0000x reference)
import jax
import jax.numpy as jnp
from jax.experimental import pallas as pl
from jax.experimental.pallas import tpu as pltpu

N = 32768
D = 4096
E = 64
TAU = 1.0

TILE = 256


def _router_kernel(x_ref, wt_ref, b_ref, idx_ref, w_ref, probs_ref, logits_ref):
    x = x_ref[...]
    wt = wt_ref[...]
    logits = jax.lax.dot_general(
        x, wt, (((1,), (0,)), ((), ())),
        preferred_element_type=jnp.float32,
    )
    logits = logits + b_ref[...]
    inv_tau = 1.0 / max(TAU, 1e-06)
    if inv_tau != 1.0:
        logits = logits * inv_tau
    logits_ref[...] = logits

    iota = jax.lax.broadcasted_iota(jnp.int32, logits.shape, 1)
    m1 = jnp.max(logits, axis=1, keepdims=True)
    idx1 = jnp.min(jnp.where(logits == m1, iota, E), axis=1, keepdims=True)
    masked = jnp.where(iota == idx1, -jnp.inf, logits)
    m2 = jnp.max(masked, axis=1, keepdims=True)
    idx2 = jnp.min(jnp.where(masked == m2, iota, E), axis=1, keepdims=True)

    idx_ref[...] = jnp.concatenate([idx1, idx2], axis=1)

    w1 = 1.0 / (1.0 + jnp.exp(m2 - m1))
    w_ref[...] = jnp.concatenate([w1, 1.0 - w1], axis=1)

    p = jnp.exp(logits - m1)
    probs_ref[...] = p / jnp.sum(p, axis=1, keepdims=True)


@jax.jit
def kernel(x, W, b):
    wt = W.T
    b2 = b.reshape(1, E)
    grid = (N // TILE,)
    out_shapes = (
        jax.ShapeDtypeStruct((N, 2), jnp.int32),
        jax.ShapeDtypeStruct((N, 2), jnp.float32),
        jax.ShapeDtypeStruct((N, E), jnp.float32),
        jax.ShapeDtypeStruct((N, E), jnp.float32),
    )
    row_spec2 = pl.BlockSpec((TILE, 2), lambda i: (i, 0))
    row_specE = pl.BlockSpec((TILE, E), lambda i: (i, 0))
    top_idx, top_w, probs_full, logits = pl.pallas_call(
        _router_kernel,
        grid=grid,
        in_specs=[
            pl.BlockSpec((TILE, D), lambda i: (i, 0)),
            pl.BlockSpec((D, E), lambda i: (0, 0)),
            pl.BlockSpec((1, E), lambda i: (0, 0)),
        ],
        out_specs=(row_spec2, row_spec2, row_specE, row_specE),
        out_shape=out_shapes,
        compiler_params=pltpu.CompilerParams(
            dimension_semantics=("arbitrary",),
        ),
    )(x, wt, b2)
    return (top_idx, top_w, probs_full, logits)


# speedup vs baseline: 1.2645x; 1.2645x over previous
import jax
import jax.numpy as jnp
from jax.experimental import pallas as pl
from jax.experimental.pallas import tpu as pltpu

N = 32768
D = 4096
E = 64
TAU = 1.0

TILE = 1024


def _router_kernel(x_ref, wt_ref, b_ref, idx_ref, w_ref, probs_ref, logits_ref):
    x = x_ref[...]
    wt = wt_ref[...]
    logits = jax.lax.dot_general(
        x, wt, (((1,), (0,)), ((), ())),
        preferred_element_type=jnp.float32,
    )
    logits = logits + b_ref[...]
    inv_tau = 1.0 / max(TAU, 1e-06)
    if inv_tau != 1.0:
        logits = logits * inv_tau
    logits_ref[...] = logits

    iota = jax.lax.broadcasted_iota(jnp.int32, logits.shape, 1)
    m1 = jnp.max(logits, axis=1, keepdims=True)
    idx1 = jnp.min(jnp.where(logits == m1, iota, E), axis=1, keepdims=True)
    masked = jnp.where(iota == idx1, -jnp.inf, logits)
    m2 = jnp.max(masked, axis=1, keepdims=True)
    idx2 = jnp.min(jnp.where(masked == m2, iota, E), axis=1, keepdims=True)

    idx_ref[...] = jnp.concatenate([idx1, idx2], axis=1)

    w1 = 1.0 / (1.0 + jnp.exp(m2 - m1))
    w_ref[...] = jnp.concatenate([w1, 1.0 - w1], axis=1)

    p = jnp.exp(logits - m1)
    probs_ref[...] = p / jnp.sum(p, axis=1, keepdims=True)


@jax.jit
def kernel(x, W, b):
    wt = W.T
    b2 = b.reshape(1, E)
    grid = (N // TILE,)
    out_shapes = (
        jax.ShapeDtypeStruct((N, 2), jnp.int32),
        jax.ShapeDtypeStruct((N, 2), jnp.float32),
        jax.ShapeDtypeStruct((N, E), jnp.float32),
        jax.ShapeDtypeStruct((N, E), jnp.float32),
    )
    row_spec2 = pl.BlockSpec((TILE, 2), lambda i: (i, 0))
    row_specE = pl.BlockSpec((TILE, E), lambda i: (i, 0))
    top_idx, top_w, probs_full, logits = pl.pallas_call(
        _router_kernel,
        grid=grid,
        in_specs=[
            pl.BlockSpec((TILE, D), lambda i: (i, 0)),
            pl.BlockSpec((D, E), lambda i: (0, 0)),
            pl.BlockSpec((1, E), lambda i: (0, 0)),
        ],
        out_specs=(row_spec2, row_spec2, row_specE, row_specE),
        out_shape=out_shapes,
        compiler_params=pltpu.CompilerParams(
            dimension_semantics=("arbitrary",),
        ),
    )(x, wt, b2)
    return (top_idx, top_w, probs_full, logits)


# speedup vs baseline: 1.3057x; 1.0326x over previous
import jax
import jax.numpy as jnp
from jax.experimental import pallas as pl
from jax.experimental.pallas import tpu as pltpu

N = 32768
D = 4096
E = 64
TAU = 1.0

TILE = 1024


def _router_kernel(x_ref, w_ref_in, b_ref, idx_ref, w_ref, probs_ref, logits_ref):
    x = x_ref[...]
    w = w_ref_in[...]
    logits = jax.lax.dot_general(
        x, w, (((1,), (1,)), ((), ())),
        preferred_element_type=jnp.float32,
    )
    logits = logits + b_ref[...]
    inv_tau = 1.0 / max(TAU, 1e-06)
    if inv_tau != 1.0:
        logits = logits * inv_tau
    logits_ref[...] = logits

    iota = jax.lax.broadcasted_iota(jnp.int32, logits.shape, 1)
    m1 = jnp.max(logits, axis=1, keepdims=True)
    idx1 = jnp.min(jnp.where(logits == m1, iota, E), axis=1, keepdims=True)
    masked = jnp.where(iota == idx1, -jnp.inf, logits)
    m2 = jnp.max(masked, axis=1, keepdims=True)
    idx2 = jnp.min(jnp.where(masked == m2, iota, E), axis=1, keepdims=True)

    idx_ref[...] = jnp.concatenate([idx1, idx2], axis=1)

    w1 = 1.0 / (1.0 + jnp.exp(m2 - m1))
    w_ref[...] = jnp.concatenate([w1, 1.0 - w1], axis=1)

    p = jnp.exp(logits - m1)
    probs_ref[...] = p / jnp.sum(p, axis=1, keepdims=True)


@jax.jit
def kernel(x, W, b):
    b2 = b.reshape(1, E)
    grid = (N // TILE,)
    out_shapes = (
        jax.ShapeDtypeStruct((N, 2), jnp.int32),
        jax.ShapeDtypeStruct((N, 2), jnp.float32),
        jax.ShapeDtypeStruct((N, E), jnp.float32),
        jax.ShapeDtypeStruct((N, E), jnp.float32),
    )
    row_spec2 = pl.BlockSpec((TILE, 2), lambda i: (i, 0))
    row_specE = pl.BlockSpec((TILE, E), lambda i: (i, 0))
    top_idx, top_w, probs_full, logits = pl.pallas_call(
        _router_kernel,
        grid=grid,
        in_specs=[
            pl.BlockSpec((TILE, D), lambda i: (i, 0)),
            pl.BlockSpec((E, D), lambda i: (0, 0)),
            pl.BlockSpec((1, E), lambda i: (0, 0)),
        ],
        out_specs=(row_spec2, row_spec2, row_specE, row_specE),
        out_shape=out_shapes,
        compiler_params=pltpu.CompilerParams(
            dimension_semantics=("arbitrary",),
        ),
    )(x, W, b2)
    return (top_idx, top_w, probs_full, logits)
